# baseline (speedup 1.0000x reference)
import functools

import jax
import jax.numpy as jnp
from jax import lax
from jax.experimental import pallas as pl
from jax.experimental.pallas import tpu as pltpu
from jax.experimental.pallas import tpu_sc as plsc

HIDDEN = 512
B = 16384
NFEAT = 47
VOCAB = 128
EMB = HIDDEN // 4

NW = 32
BT = B // NW
LIN_BLK = 512


def _sc_disc_body(tab_hbm, idx_hbm, out_hbm, tab_v, idx_v, out_v):
    wid = lax.axis_index("s") * 2 + lax.axis_index("c")
    base = wid * BT
    pltpu.sync_copy(tab_hbm, tab_v)
    pltpu.sync_copy(idx_hbm.at[pl.ds(base * NFEAT, BT * NFEAT)],
                    idx_v.at[pl.ds(0, BT * NFEAT)])

    def pbody(p, carry):
        pf = p * NFEAT
        iv = [idx_v[pl.ds(pf + 16 * k, 16)] for k in range(3)]
        idxs = [iv[k][j] for k in range(3) for j in range(16)][:NFEAT]
        offs = [idx * EMB for idx in idxs]
        grp = [[], [], [], []]
        for f in range(NFEAT):
            grp[f % 4].append(offs[f])

        def row(off, u):
            return tab_v[pl.ds(off + 16 * u, 16)]

        po = p * EMB
        for u in range(8):
            gacc = []
            for g in range(4):
                a = row(grp[g][0], u)
                for off in grp[g][1:]:
                    a = jnp.maximum(a, row(off, u))
                gacc.append(a)
            acc = jnp.maximum(jnp.maximum(gacc[0], gacc[1]),
                              jnp.maximum(gacc[2], gacc[3]))
            out_v[pl.ds(po + 16 * u, 16)] = acc
        return carry

    lax.fori_loop(0, BT, pbody, 0)
    pltpu.sync_copy(out_v, out_hbm.at[pl.ds(base * EMB, BT * EMB)])


def _sc_disc(tab_pk, player_flat):
    mesh = plsc.VectorSubcoreMesh(core_axis_name="c", subcore_axis_name="s")
    k = functools.partial(
        pl.kernel,
        mesh=mesh,
        out_type=jax.ShapeDtypeStruct((B * EMB,), jnp.float32),
        scratch_types=[
            pltpu.VMEM((VOCAB * EMB,), jnp.float32),
            pltpu.VMEM((BT * NFEAT + 16,), jnp.int32),
            pltpu.VMEM((BT * EMB,), jnp.float32),
        ],
    )(_sc_disc_body)
    return k(tab_pk, player_flat)


def _lin_body(p_ref, w_ref, b_ref, o_ref):
    x = p_ref[...].astype(jnp.float32) / 99.0
    o_ref[...] = jax.lax.dot_general(
        x, w_ref[...], (((1,), (0,)), ((), ())),
        preferred_element_type=jnp.float32) + b_ref[...]


def _tc_linear(player_mod, W_cont, b_cont):
    wT = W_cont.T
    b2 = b_cont.reshape(1, EMB)
    return pl.pallas_call(
        _lin_body,
        grid=(B // LIN_BLK,),
        in_specs=[
            pl.BlockSpec((LIN_BLK, NFEAT), lambda i: (i, 0)),
            pl.BlockSpec((NFEAT, EMB), lambda i: (0, 0)),
            pl.BlockSpec((1, EMB), lambda i: (0, 0)),
        ],
        out_specs=pl.BlockSpec((LIN_BLK, EMB), lambda i: (i, 0)),
        out_shape=jax.ShapeDtypeStruct((B, EMB), jnp.float32),
    )(player_mod, wT, b2)


def kernel(player, embed_table, W_cont, b_cont):
    player_mod = player.at[-2:].set(player[-2:] // 10)
    disc = _sc_disc(embed_table.reshape(VOCAB * EMB),
                    player_mod.reshape(B * NFEAT)).reshape(B, EMB)
    cont = _tc_linear(player_mod, W_cont, b_cont)
    return jnp.concatenate([disc, cont], axis=1)


# speedup vs baseline: 1.1347x; 1.1347x over previous
import functools

import jax
import jax.numpy as jnp
from jax import lax
from jax.experimental import pallas as pl
from jax.experimental.pallas import tpu as pltpu
from jax.experimental.pallas import tpu_sc as plsc

HIDDEN = 512
B = 16384
NFEAT = 47
VOCAB = 128
EMB = HIDDEN // 4
ROWW = 8

NW = 32
B_SC = B
BT = B_SC // NW
TC_BLK = 256


def _sc_disc_body(tab_hbm, idx_hbm, out_hbm, tab_v, idx_v, out_v):
    wid = lax.axis_index("s") * 2 + lax.axis_index("c")
    base = wid * BT
    pltpu.sync_copy(tab_hbm, tab_v)
    pltpu.sync_copy(idx_hbm.at[pl.ds(base * NFEAT, BT * NFEAT)],
                    idx_v.at[pl.ds(0, BT * NFEAT)])

    def pbody(p, carry):
        pf = p * NFEAT
        iv = [idx_v[pl.ds(pf + 16 * k, 16)] for k in range(3)]
        idxs = [iv[k][j] for k in range(3) for j in range(16)][:NFEAT]
        rows = [idx * 2 for idx in idxs]
        grp = [[], [], [], []]
        for f in range(NFEAT):
            grp[f % 4].append(rows[f])

        po = p * 2
        for u in range(4):
            gacc = []
            for g in range(4):
                a = tab_v[pl.ds(grp[g][0], 2), pl.ds(16 * u, 16)]
                for r in grp[g][1:]:
                    a = jnp.maximum(
                        a, tab_v[pl.ds(r, 2), pl.ds(16 * u, 16)])
                gacc.append(a)
            acc = jnp.maximum(jnp.maximum(gacc[0], gacc[1]),
                              jnp.maximum(gacc[2], gacc[3]))
            out_v[pl.ds(po, 2), pl.ds(16 * u, 16)] = acc
        return carry

    lax.fori_loop(0, BT, pbody, 0)
    pltpu.sync_copy(out_v, out_hbm.at[pl.ds(base * 2, BT * 2)])


def _sc_disc(tab_rows, player_flat):
    mesh = plsc.VectorSubcoreMesh(core_axis_name="c", subcore_axis_name="s")
    k = functools.partial(
        pl.kernel,
        mesh=mesh,
        out_type=jax.ShapeDtypeStruct((B_SC * 2, EMB // 2), jnp.bfloat16),
        scratch_types=[
            pltpu.VMEM((VOCAB * 2, EMB // 2), jnp.bfloat16),
            pltpu.VMEM((BT * NFEAT + 16,), jnp.int32),
            pltpu.VMEM((BT * 2, EMB // 2), jnp.bfloat16),
        ],
    )(_sc_disc_body)
    return k(tab_rows, player_flat)


def _asm_body(d_ref, p_ref, w_ref, b_ref, o_ref):
    o_ref[:, :EMB] = d_ref[...].astype(jnp.float32)
    x = p_ref[...].astype(jnp.float32) / 99.0
    o_ref[:, EMB:] = jax.lax.dot_general(
        x, w_ref[...], (((1,), (0,)), ((), ())),
        preferred_element_type=jnp.float32) + b_ref[...]


def _tc_assemble(disc_bf, player_mod, wT, b2):
    return pl.pallas_call(
        _asm_body,
        grid=(B_SC // TC_BLK,),
        in_specs=[
            pl.BlockSpec((TC_BLK, EMB), lambda i: (i, 0)),
            pl.BlockSpec((TC_BLK, NFEAT), lambda i: (i, 0)),
            pl.BlockSpec((NFEAT, EMB), lambda i: (0, 0)),
            pl.BlockSpec((1, EMB), lambda i: (0, 0)),
        ],
        out_specs=pl.BlockSpec((TC_BLK, 2 * EMB), lambda i: (i, 0)),
        out_shape=jax.ShapeDtypeStruct((B_SC, 2 * EMB), jnp.float32),
    )(disc_bf, player_mod, wT, b2)


def _tc_body(p_ref, tab_ref, w_ref, b_ref, o_ref):
    p = p_ref[...]
    tab = tab_ref[...]
    acc = jnp.full((TC_BLK, EMB), -jnp.inf, jnp.float32)
    for f in range(NFEAT):
        col = jax.lax.slice(p, (0, f), (TC_BLK, f + 1))
        oh = (col == jax.lax.broadcasted_iota(jnp.int32, (TC_BLK, VOCAB), 1)
              ).astype(jnp.bfloat16)
        emb_f = jax.lax.dot_general(
            oh, tab, (((1,), (0,)), ((), ())),
            preferred_element_type=jnp.float32)
        acc = jnp.maximum(acc, emb_f)
    x = p.astype(jnp.float32) / 99.0
    cont = jax.lax.dot_general(
        x, w_ref[...], (((1,), (0,)), ((), ())),
        preferred_element_type=jnp.float32) + b_ref[...]
    o_ref[:, :EMB] = acc
    o_ref[:, EMB:] = cont


def _tc_tail(player_tail, tab_bf, wT, b2):
    n = B - B_SC
    return pl.pallas_call(
        _tc_body,
        grid=(n // TC_BLK,),
        in_specs=[
            pl.BlockSpec((TC_BLK, NFEAT), lambda i: (i, 0)),
            pl.BlockSpec((VOCAB, EMB), lambda i: (0, 0)),
            pl.BlockSpec((NFEAT, EMB), lambda i: (0, 0)),
            pl.BlockSpec((1, EMB), lambda i: (0, 0)),
        ],
        out_specs=pl.BlockSpec((TC_BLK, 2 * EMB), lambda i: (i, 0)),
        out_shape=jax.ShapeDtypeStruct((n, 2 * EMB), jnp.float32),
    )(player_tail, tab_bf, wT, b2)


def kernel(player, embed_table, W_cont, b_cont):
    player_mod = player.at[-2:].set(player[-2:] // 10)
    tab_bf = embed_table.astype(jnp.bfloat16)
    wT = W_cont.T
    b2 = b_cont.reshape(1, EMB)

    disc_bf = _sc_disc(tab_bf.reshape(VOCAB * 2, EMB // 2),
                       player_mod[:B_SC].reshape(B_SC * NFEAT))
    head = _tc_assemble(disc_bf.reshape(B_SC, EMB), player_mod[:B_SC], wT, b2)
    if B_SC == B:
        return head
    tail = _tc_tail(player_mod[B_SC:], tab_bf, wT, b2)
    return jnp.concatenate([head, tail], axis=0)


# speedup vs baseline: 1.3958x; 1.2302x over previous
import functools

import jax
import jax.numpy as jnp
from jax import lax
from jax.experimental import pallas as pl
from jax.experimental.pallas import tpu as pltpu
from jax.experimental.pallas import tpu_sc as plsc

HIDDEN = 512
B = 16384
NFEAT = 47
VOCAB = 128
EMB = HIDDEN // 4
ROWW = 8

NW = 32
B_SC = 10240
BT = B_SC // NW
TC_BLK = 256


def _sc_disc_body(tab_hbm, idx_hbm, out_hbm, tab_v, idx_v, out_v):
    wid = lax.axis_index("s") * 2 + lax.axis_index("c")
    base = wid * BT
    pltpu.sync_copy(tab_hbm, tab_v)
    pltpu.sync_copy(idx_hbm.at[pl.ds(base * NFEAT, BT * NFEAT)],
                    idx_v.at[pl.ds(0, BT * NFEAT)])

    def pbody(p, carry):
        pf = p * NFEAT
        iv = [idx_v[pl.ds(pf + 16 * k, 16)] for k in range(3)]
        idxs = [iv[k][j] for k in range(3) for j in range(16)][:NFEAT]
        rows = [idx * 2 for idx in idxs]
        grp = [[], [], [], []]
        for f in range(NFEAT):
            grp[f % 4].append(rows[f])

        po = p * 2
        for u in range(4):
            gacc = []
            for g in range(4):
                a = tab_v[pl.ds(grp[g][0], 2), pl.ds(16 * u, 16)]
                for r in grp[g][1:]:
                    a = jnp.maximum(
                        a, tab_v[pl.ds(r, 2), pl.ds(16 * u, 16)])
                gacc.append(a)
            acc = jnp.maximum(jnp.maximum(gacc[0], gacc[1]),
                              jnp.maximum(gacc[2], gacc[3]))
            out_v[pl.ds(po, 2), pl.ds(16 * u, 16)] = acc
        return carry

    lax.fori_loop(0, BT, pbody, 0)
    pltpu.sync_copy(out_v, out_hbm.at[pl.ds(base * 2, BT * 2)])


def _sc_disc(tab_rows, player_flat):
    mesh = plsc.VectorSubcoreMesh(core_axis_name="c", subcore_axis_name="s")
    k = functools.partial(
        pl.kernel,
        mesh=mesh,
        out_type=jax.ShapeDtypeStruct((B_SC * 2, EMB // 2), jnp.bfloat16),
        scratch_types=[
            pltpu.VMEM((VOCAB * 2, EMB // 2), jnp.bfloat16),
            pltpu.VMEM((BT * NFEAT + 16,), jnp.int32),
            pltpu.VMEM((BT * 2, EMB // 2), jnp.bfloat16),
        ],
    )(_sc_disc_body)
    return k(tab_rows, player_flat)


def _asm_body(d_ref, p_ref, w_ref, b_ref, o_ref):
    o_ref[:, :EMB] = d_ref[...].astype(jnp.float32)
    x = p_ref[...].astype(jnp.float32) / 99.0
    o_ref[:, EMB:] = jax.lax.dot_general(
        x, w_ref[...], (((1,), (0,)), ((), ())),
        preferred_element_type=jnp.float32) + b_ref[...]


def _tc_assemble(disc_bf, player_mod, wT, b2):
    return pl.pallas_call(
        _asm_body,
        grid=(B_SC // TC_BLK,),
        in_specs=[
            pl.BlockSpec((TC_BLK, EMB), lambda i: (i, 0)),
            pl.BlockSpec((TC_BLK, NFEAT), lambda i: (i, 0)),
            pl.BlockSpec((NFEAT, EMB), lambda i: (0, 0)),
            pl.BlockSpec((1, EMB), lambda i: (0, 0)),
        ],
        out_specs=pl.BlockSpec((TC_BLK, 2 * EMB), lambda i: (i, 0)),
        out_shape=jax.ShapeDtypeStruct((B_SC, 2 * EMB), jnp.float32),
    )(disc_bf, player_mod, wT, b2)


def _tc_body(p_ref, tab_ref, w_ref, b_ref, o_ref):
    p = p_ref[...]
    tab = tab_ref[...]
    acc = jnp.full((TC_BLK, EMB), -jnp.inf, jnp.float32)
    for f in range(NFEAT):
        col = jax.lax.slice(p, (0, f), (TC_BLK, f + 1))
        oh = (col == jax.lax.broadcasted_iota(jnp.int32, (TC_BLK, VOCAB), 1)
              ).astype(jnp.bfloat16)
        emb_f = jax.lax.dot_general(
            oh, tab, (((1,), (0,)), ((), ())),
            preferred_element_type=jnp.float32)
        acc = jnp.maximum(acc, emb_f)
    x = p.astype(jnp.float32) / 99.0
    cont = jax.lax.dot_general(
        x, w_ref[...], (((1,), (0,)), ((), ())),
        preferred_element_type=jnp.float32) + b_ref[...]
    o_ref[:, :EMB] = acc
    o_ref[:, EMB:] = cont


def _tc_tail(player_tail, tab_bf, wT, b2):
    n = B - B_SC
    return pl.pallas_call(
        _tc_body,
        grid=(n // TC_BLK,),
        in_specs=[
            pl.BlockSpec((TC_BLK, NFEAT), lambda i: (i, 0)),
            pl.BlockSpec((VOCAB, EMB), lambda i: (0, 0)),
            pl.BlockSpec((NFEAT, EMB), lambda i: (0, 0)),
            pl.BlockSpec((1, EMB), lambda i: (0, 0)),
        ],
        out_specs=pl.BlockSpec((TC_BLK, 2 * EMB), lambda i: (i, 0)),
        out_shape=jax.ShapeDtypeStruct((n, 2 * EMB), jnp.float32),
    )(player_tail, tab_bf, wT, b2)


def kernel(player, embed_table, W_cont, b_cont):
    player_mod = player.at[-2:].set(player[-2:] // 10)
    tab_bf = embed_table.astype(jnp.bfloat16)
    wT = W_cont.T
    b2 = b_cont.reshape(1, EMB)

    disc_bf = _sc_disc(tab_bf.reshape(VOCAB * 2, EMB // 2),
                       player_mod[:B_SC].reshape(B_SC * NFEAT))
    head = _tc_assemble(disc_bf.reshape(B_SC, EMB), player_mod[:B_SC], wT, b2)
    if B_SC == B:
        return head
    tail = _tc_tail(player_mod[B_SC:], tab_bf, wT, b2)
    return jnp.concatenate([head, tail], axis=0)
